# baseline (speedup 1.0000x reference)
import functools

import jax
import jax.numpy as jnp
from jax import lax
from jax.experimental import pallas as pl
from jax.experimental.pallas import tpu as pltpu
from jax.experimental.pallas import tpu_sc as plsc

SEQ_LEN = 8192
BATCH = 2
D_MODEL = 1024

NC = 2
NS = 16
NW = NC * NS
ROWS_PER_W = SEQ_LEN // NW
TILE = 8
NT = ROWS_PER_W // TILE
NBUF = 4
PREFETCH = 2
NVEC = D_MODEL // 16


def _sc_body(x_hbm, t_hbm, out_hbm, xv, tv, sem_in_x, sem_in_t, sem_out):
    wid = lax.axis_index("s") * NC + lax.axis_index("c")
    base = wid * ROWS_PER_W

    def in_x(g, buf):
        return pltpu.make_async_copy(
            x_hbm.at[pl.ds(base + g * TILE, TILE)], xv.at[buf], sem_in_x.at[buf])

    def in_t(g, buf):
        return pltpu.make_async_copy(
            t_hbm.at[pl.ds(base + g * TILE, TILE)], tv.at[buf], sem_in_t.at[buf])

    def out_cp(g, buf):
        return pltpu.make_async_copy(
            xv.at[buf], out_hbm.at[pl.ds(base + g * TILE, TILE)], sem_out.at[buf])

    def compute(buf):
        def row_body(r, _):
            for v in range(NVEC):
                dslice = pl.ds(v * 16, 16)
                t = tv[buf, r, dslice]
                xv[buf, r, 0, dslice] = xv[buf, r, 0, dslice] + t
                xv[buf, r, 1, dslice] = xv[buf, r, 1, dslice] + t
            return 0
        lax.fori_loop(0, TILE, row_body, 0)

    for p in range(PREFETCH):
        in_x(p, p).start()
        in_t(p, p).start()

    def step(g, _):
        buf = lax.rem(g, NBUF)
        nxt = g + PREFETCH

        @pl.when(nxt < NT)
        def _():
            nbuf = lax.rem(nxt, NBUF)

            @pl.when(nxt >= NBUF)
            def _():
                out_cp(nxt - NBUF, nbuf).wait()

            in_x(nxt, nbuf).start()
            in_t(nxt, nbuf).start()

        in_x(g, buf).wait()
        in_t(g, buf).wait()
        compute(buf)
        out_cp(g, buf).start()
        return 0

    lax.fori_loop(0, NT, step, 0)

    for i in range(NBUF):
        g = NT - NBUF + i
        out_cp(g, g % NBUF).wait()


def kernel(x, table):
    f = functools.partial(
        pl.kernel,
        mesh=plsc.VectorSubcoreMesh(core_axis_name="c", subcore_axis_name="s"),
        out_type=jax.ShapeDtypeStruct((SEQ_LEN, BATCH, D_MODEL), jnp.float32),
        scratch_types=[
            pltpu.VMEM((NBUF, TILE, BATCH, D_MODEL), jnp.float32),
            pltpu.VMEM((NBUF, TILE, D_MODEL), jnp.float32),
            pltpu.SemaphoreType.DMA((NBUF,)),
            pltpu.SemaphoreType.DMA((NBUF,)),
            pltpu.SemaphoreType.DMA((NBUF,)),
        ],
    )(_sc_body)
    return f(x, table)
